# baseline (speedup 1.0000x reference)
import jax
import jax.numpy as jnp
from jax import lax
from jax.experimental import pallas as pl
from jax.experimental.pallas import tpu as pltpu


def _fused_body(ph_ref, tok_ref, emb_ref, x_ref, wv_ref, wo_ref, bo_ref,
                wn_ref, bn_ref, out_ref):
    d = x_ref.shape[1]
    n = emb_ref.shape[1] // d
    np_ = tok_ref.shape[1]
    inner = wv_ref.shape[0] * wv_ref.shape[1] // d
    x = x_ref[...]
    wv = wv_ref[...].reshape(d, inner)
    wo = wo_ref[...].reshape(inner, d)
    wn = wn_ref[...].reshape(d, d)
    t = jnp.dot(x, wv, preferred_element_type=jnp.float32)
    t = jnp.dot(t, wo, preferred_element_type=jnp.float32) + bo_ref[...]
    p = jnp.dot(t, wn, preferred_element_type=jnp.float32) + bn_ref[...]
    m = (tok_ref[...] == ph_ref[0]).astype(jnp.float32)
    rows = lax.broadcasted_iota(jnp.int32, (n, np_), 0)
    cols = lax.broadcasted_iota(jnp.int32, (n, np_), 1)
    eye = (rows == cols).astype(jnp.float32)
    maskcol = lax.dot_general(eye, m, (((1,), (1,)), ((), ())),
                              preferred_element_type=jnp.float32)
    emb = emb_ref[...].reshape(n, d)
    outv = jnp.where(maskcol > 0.5, p, emb)
    out_ref[...] = outv.reshape(1, n * d)


def kernel(tokenized_text, embedded_text, image_embeds, placeholder_token,
           Wq1, Wk1, Wv1, Wo1, bo1, Wq2, Wk2, Wv2, Wo2, bo2, Wn, bn):
    b, n = tokenized_text.shape
    d = embedded_text.shape[-1]
    npad = (n + 7) // 8 * 8
    tok = jnp.pad(tokenized_text.reshape(1, n), ((0, 0), (0, npad - n)))
    emb = embedded_text.reshape(1, n * d)
    x = image_embeds.reshape(1, d)
    ph = placeholder_token.reshape(1)
    out = pl.pallas_call(
        _fused_body,
        out_shape=jax.ShapeDtypeStruct((1, n * d), jnp.float32),
        in_specs=[
            pl.BlockSpec(memory_space=pltpu.SMEM),
            pl.BlockSpec(memory_space=pltpu.VMEM),
            pl.BlockSpec(memory_space=pltpu.VMEM),
            pl.BlockSpec(memory_space=pltpu.VMEM),
            pl.BlockSpec(memory_space=pltpu.VMEM),
            pl.BlockSpec(memory_space=pltpu.VMEM),
            pl.BlockSpec(memory_space=pltpu.VMEM),
            pl.BlockSpec(memory_space=pltpu.VMEM),
            pl.BlockSpec(memory_space=pltpu.VMEM),
        ],
        out_specs=pl.BlockSpec(memory_space=pltpu.VMEM),
    )(ph, tok, emb, x,
      Wv2.reshape(Wv2.shape[0] // 8, -1),
      Wo2.reshape(Wo2.shape[0] // 8, -1),
      bo2.reshape(1, d),
      Wn.reshape(Wn.shape[0] // 8, -1),
      bn.reshape(1, d))
    return out.reshape(b, n, d)


# speedup vs baseline: 2.3196x; 2.3196x over previous
import jax
import jax.numpy as jnp
from jax import lax
from jax.experimental import pallas as pl
from jax.experimental.pallas import tpu as pltpu


def _fused_body(ph_ref, tok_ref, emb_ref, x_ref, wv_ref, wo_ref, bo_ref,
                wn_ref, bn_ref, out_ref):
    d = x_ref.shape[1]
    n = emb_ref.shape[1] // d
    np_ = tok_ref.shape[1]
    x = x_ref[...]
    t = jnp.dot(x, wv_ref[...], preferred_element_type=jnp.float32)
    t = jnp.dot(t, wo_ref[...], preferred_element_type=jnp.float32) + bo_ref[...]
    p = jnp.dot(t, wn_ref[...], preferred_element_type=jnp.float32) + bn_ref[...]
    m = (tok_ref[...] == ph_ref[0]).astype(jnp.float32)
    rows = lax.broadcasted_iota(jnp.int32, (n, np_), 0)
    cols = lax.broadcasted_iota(jnp.int32, (n, np_), 1)
    eye = (rows == cols).astype(jnp.float32)
    maskcol = lax.dot_general(eye, m, (((1,), (1,)), ((), ())),
                              preferred_element_type=jnp.float32)
    emb = emb_ref[...].reshape(n, d)
    outv = jnp.where(maskcol > 0.5, p, emb)
    out_ref[...] = outv.reshape(1, n * d)


def kernel(tokenized_text, embedded_text, image_embeds, placeholder_token,
           Wq1, Wk1, Wv1, Wo1, bo1, Wq2, Wk2, Wv2, Wo2, bo2, Wn, bn):
    b, n = tokenized_text.shape
    d = embedded_text.shape[-1]
    tok = tokenized_text.reshape(1, n)
    emb = embedded_text.reshape(1, n * d)
    x = image_embeds.reshape(1, d)
    ph = placeholder_token.reshape(1)
    out = pl.pallas_call(
        _fused_body,
        out_shape=jax.ShapeDtypeStruct((1, n * d), jnp.float32),
        in_specs=[
            pl.BlockSpec(memory_space=pltpu.SMEM),
            pl.BlockSpec(memory_space=pltpu.VMEM),
            pl.BlockSpec(memory_space=pltpu.VMEM),
            pl.BlockSpec(memory_space=pltpu.VMEM),
            pl.BlockSpec(memory_space=pltpu.VMEM),
            pl.BlockSpec(memory_space=pltpu.VMEM),
            pl.BlockSpec(memory_space=pltpu.VMEM),
            pl.BlockSpec(memory_space=pltpu.VMEM),
            pl.BlockSpec(memory_space=pltpu.VMEM),
        ],
        out_specs=pl.BlockSpec(memory_space=pltpu.VMEM),
    )(ph, tok, emb, x, Wv2, Wo2, bo2.reshape(1, d), Wn, bn.reshape(1, d))
    return out.reshape(b, n, d)
